# Initial kernel scaffold; baseline (speedup 1.0000x reference)
import jax
import jax.numpy as jnp
from jax.experimental import pallas as pl


def kernel(x, edge_index, W1, b1, W2, b2, gamma, beta, l1W, l1b, l2W, l2b, l3W, l3b):
    raise NotImplementedError("write your pallas kernel here")


# speedup vs baseline: 14.5633x; 14.5633x over previous
import functools

import jax
import jax.numpy as jnp
from jax.experimental import pallas as pl
from jax.experimental.pallas import tpu as pltpu

N_NODES = 100
N_EDGES = 3200
E_PAD = 3328
NP = 128
EPS = 1e-5


def _rsqrt(v):
    r = jax.lax.rsqrt(v)
    r = r * (1.5 - 0.5 * v * r * r)
    r = r * (1.5 - 0.5 * v * r * r)
    return r


def _bn(h, gamma, beta):
    inv_n = 1.0 / N_NODES
    mean = jnp.sum(h, axis=0, keepdims=True) * inv_n
    xc = h - mean
    var = jnp.sum(xc * xc, axis=0, keepdims=True) * inv_n
    return xc * _rsqrt(var + EPS) * gamma + beta


def _gcn_kernel(ei_ref, x_ref, w1_ref, b1_ref, w2_ref, b2_ref, gamma_ref,
                beta_ref, l1w_ref, l1b_ref, l2w_ref, l2b_ref, l3w_ref,
                l3b_ref, out_ref):
    f32 = jnp.float32
    bf = jnp.bfloat16
    srcv = ei_ref[0:1, :]
    dstv = ei_ref[1:2, :]
    jrow = jax.lax.broadcasted_iota(jnp.int32, (NP, E_PAD), 0)
    st = (jrow == srcv).astype(bf)
    dt = (jrow == dstv).astype(bf)

    cnt = jax.lax.dot_general(dt, st, (((1,), (1,)), ((), ())),
                              preferred_element_type=f32)
    deg = jnp.sum(cnt, axis=1, keepdims=True)
    dis_c = jnp.where(deg > 0, _rsqrt(jnp.maximum(deg, 1.0)), 0.0)
    ii = jax.lax.broadcasted_iota(jnp.int32, (NP, NP), 0)
    jj = jax.lax.broadcasted_iota(jnp.int32, (NP, NP), 1)
    dis_r = jnp.sum(jnp.where(ii == jj, dis_c, 0.0), axis=0, keepdims=True)
    a = (cnt * dis_c * dis_r)[:N_NODES, :N_NODES]

    hi = jax.lax.Precision.HIGHEST

    xw1 = jnp.dot(x_ref[...].astype(bf), w1_ref[...].astype(bf),
                  preferred_element_type=f32)
    h = jnp.dot(a, xw1, preferred_element_type=f32, precision=hi) + b1_ref[...]
    h = _bn(jax.nn.relu(h), gamma_ref[...], beta_ref[...])

    xw2 = jnp.dot(h.astype(bf), w2_ref[...].astype(bf),
                  preferred_element_type=f32)
    h = jnp.dot(a, xw2, preferred_element_type=f32, precision=hi) + b2_ref[...]
    h = _bn(jax.nn.relu(h), gamma_ref[...], beta_ref[...])

    prod = h.astype(bf).astype(f32)[:, :, None] * l1w_ref[...].astype(bf).astype(f32)
    fc1 = jnp.sum(jnp.sum(prod, axis=0), axis=0, keepdims=True)
    r = jax.nn.relu(fc1 + l1b_ref[...])
    r = jax.nn.relu(jnp.dot(r.astype(bf), l2w_ref[...].astype(bf),
                            preferred_element_type=f32) + l2b_ref[...])
    out_ref[...] = (jnp.dot(r.astype(bf), l3w_ref[...].astype(bf),
                            preferred_element_type=f32) + l3b_ref[...])


@functools.partial(jax.jit, static_argnames=())
def kernel(x, edge_index, W1, b1, W2, b2, gamma, beta, l1W, l1b, l2W, l2b,
           l3W, l3b):
    loops = jnp.broadcast_to(jnp.arange(N_NODES, dtype=edge_index.dtype),
                             (2, N_NODES))
    pad = jnp.full((2, E_PAD - N_EDGES - N_NODES), -1, edge_index.dtype)
    ei = jnp.concatenate([edge_index, loops, pad], axis=1).astype(jnp.int32)

    out = pl.pallas_call(
        _gcn_kernel,
        out_shape=jax.ShapeDtypeStruct((1, 2), jnp.float32),
    )(
        ei, x, W1, b1.reshape(1, -1), W2, b2.reshape(1, -1),
        gamma.reshape(1, -1), beta.reshape(1, -1),
        l1W.reshape(N_NODES, 64, 64), l1b.reshape(1, -1),
        l2W, l2b.reshape(1, -1), l3W, l3b.reshape(1, -1),
    )
    return out


# speedup vs baseline: 16.4483x; 1.1294x over previous
import functools

import jax
import jax.numpy as jnp
from jax.experimental import pallas as pl
from jax.experimental.pallas import tpu as pltpu

N_NODES = 100
N_EDGES = 3200
E_PAD = 3328
NP = 128
EPS = 1e-5


def _rsqrt(v):
    r = jax.lax.rsqrt(v)
    r = r * (1.5 - 0.5 * v * r * r)
    r = r * (1.5 - 0.5 * v * r * r)
    return r


def _bn(h, gamma, beta):
    inv_n = 1.0 / N_NODES
    mean = jnp.sum(h, axis=0, keepdims=True) * inv_n
    xc = h - mean
    var = jnp.sum(xc * xc, axis=0, keepdims=True) * inv_n
    return xc * _rsqrt(var + EPS) * gamma + beta


def _gcn_kernel(ei_ref, x_ref, w1_ref, b1_ref, w2_ref, b2_ref, gamma_ref,
                beta_ref, l1w_ref, l1b_ref, l2w_ref, l2b_ref, l3w_ref,
                l3b_ref, out_ref):
    f32 = jnp.float32
    bf = jnp.bfloat16
    srcv = ei_ref[0:1, :]
    dstv = ei_ref[1:2, :]
    jrow = jax.lax.broadcasted_iota(jnp.int32, (NP, N_EDGES), 0)
    st = (jrow == srcv).astype(bf)
    dt = (jrow == dstv).astype(bf)

    ii = jax.lax.broadcasted_iota(jnp.int32, (NP, NP), 0)
    jj = jax.lax.broadcasted_iota(jnp.int32, (NP, NP), 1)
    eye = ((ii == jj) & (ii < N_NODES)).astype(f32)
    cnt = jax.lax.dot_general(dt, st, (((1,), (1,)), ((), ())),
                              preferred_element_type=f32) + eye
    deg = jnp.sum(cnt, axis=1, keepdims=True)
    dis_c = jnp.where(deg > 0, _rsqrt(jnp.maximum(deg, 1.0)), 0.0)
    dis_r = jnp.sum(jnp.where(ii == jj, dis_c, 0.0), axis=0, keepdims=True)
    a = (cnt * dis_c * dis_r)[:N_NODES, :N_NODES]

    hi = jax.lax.Precision.HIGHEST

    xw1 = jnp.dot(x_ref[...].astype(bf), w1_ref[...].astype(bf),
                  preferred_element_type=f32)
    h = jnp.dot(a, xw1, preferred_element_type=f32, precision=hi) + b1_ref[...]
    h = _bn(jax.nn.relu(h), gamma_ref[...], beta_ref[...])

    xw2 = jnp.dot(h.astype(bf), w2_ref[...].astype(bf),
                  preferred_element_type=f32)
    h = jnp.dot(a, xw2, preferred_element_type=f32, precision=hi) + b2_ref[...]
    h = _bn(jax.nn.relu(h), gamma_ref[...], beta_ref[...])

    prod = h.astype(bf).astype(f32)[:, :, None] * l1w_ref[...].astype(bf).astype(f32)
    fc1 = jnp.sum(jnp.sum(prod, axis=0), axis=0, keepdims=True)
    r = jax.nn.relu(fc1 + l1b_ref[...])
    r = jax.nn.relu(jnp.dot(r.astype(bf), l2w_ref[...].astype(bf),
                            preferred_element_type=f32) + l2b_ref[...])
    out_ref[...] = (jnp.dot(r.astype(bf), l3w_ref[...].astype(bf),
                            preferred_element_type=f32) + l3b_ref[...])


@functools.partial(jax.jit, static_argnames=())
def kernel(x, edge_index, W1, b1, W2, b2, gamma, beta, l1W, l1b, l2W, l2b,
           l3W, l3b):
    ei = edge_index.astype(jnp.int32)

    out = pl.pallas_call(
        _gcn_kernel,
        out_shape=jax.ShapeDtypeStruct((1, 2), jnp.float32),
    )(
        ei, x, W1, b1.reshape(1, -1), W2, b2.reshape(1, -1),
        gamma.reshape(1, -1), beta.reshape(1, -1),
        l1W.reshape(N_NODES, 64, 64), l1b.reshape(1, -1),
        l2W, l2b.reshape(1, -1), l3W, l3b.reshape(1, -1),
    )
    return out


# speedup vs baseline: 17.3909x; 1.0573x over previous
import functools

import jax
import jax.numpy as jnp
from jax.experimental import pallas as pl
from jax.experimental.pallas import tpu as pltpu

N_NODES = 100
N_EDGES = 3200
E_PAD = 3328
NP = 128
EPS = 1e-5


def _rsqrt(v):
    r = jax.lax.rsqrt(v)
    r = r * (1.5 - 0.5 * v * r * r)
    r = r * (1.5 - 0.5 * v * r * r)
    return r


def _bn(h, gamma, beta):
    inv_n = 1.0 / N_NODES
    mean = jnp.sum(h, axis=0, keepdims=True) * inv_n
    xc = h - mean
    var = jnp.sum(xc * xc, axis=0, keepdims=True) * inv_n
    return xc * _rsqrt(var + EPS) * gamma + beta


def _gcn_kernel(ei_ref, x_ref, w1_ref, b1_ref, w2_ref, b2_ref, gamma_ref,
                beta_ref, l1w_hbm_ref, l1b_ref, l2w_ref, l2b_ref, l3w_ref,
                l3b_ref, out_ref, l1w_ref, dma_sem):
    f32 = jnp.float32
    bf = jnp.bfloat16
    l1w_copy = pltpu.make_async_copy(l1w_hbm_ref, l1w_ref, dma_sem)
    l1w_copy.start()
    srcv = ei_ref[0:1, :]
    dstv = ei_ref[1:2, :]
    jrow = jax.lax.broadcasted_iota(jnp.int32, (NP, N_EDGES), 0)
    st = (jrow == srcv).astype(bf)
    dt = (jrow == dstv).astype(bf)

    ii = jax.lax.broadcasted_iota(jnp.int32, (NP, NP), 0)
    jj = jax.lax.broadcasted_iota(jnp.int32, (NP, NP), 1)
    eye = ((ii == jj) & (ii < N_NODES)).astype(f32)
    cnt = jax.lax.dot_general(dt, st, (((1,), (1,)), ((), ())),
                              preferred_element_type=f32) + eye
    deg = jnp.sum(cnt, axis=1, keepdims=True)
    dis_c = jnp.where(deg > 0, _rsqrt(jnp.maximum(deg, 1.0)), 0.0)
    dis_r = jnp.sum(jnp.where(ii == jj, dis_c, 0.0), axis=0, keepdims=True)
    a = (cnt * dis_c * dis_r)[:N_NODES, :N_NODES]

    hi = jax.lax.Precision.HIGHEST

    xw1 = jnp.dot(x_ref[...].astype(bf), w1_ref[...].astype(bf),
                  preferred_element_type=f32)
    h = jnp.dot(a, xw1, preferred_element_type=f32, precision=hi) + b1_ref[...]
    h = _bn(jax.nn.relu(h), gamma_ref[...], beta_ref[...])

    xw2 = jnp.dot(h.astype(bf), w2_ref[...].astype(bf),
                  preferred_element_type=f32)
    h = jnp.dot(a, xw2, preferred_element_type=f32, precision=hi) + b2_ref[...]
    h = _bn(jax.nn.relu(h), gamma_ref[...], beta_ref[...])

    l1w_copy.wait()
    prod = h.astype(bf).astype(f32)[:, :, None] * l1w_ref[...].astype(bf).astype(f32)
    fc1 = jnp.sum(jnp.sum(prod, axis=0), axis=0, keepdims=True)
    r = jax.nn.relu(fc1 + l1b_ref[...])
    r = jax.nn.relu(jnp.dot(r.astype(bf), l2w_ref[...].astype(bf),
                            preferred_element_type=f32) + l2b_ref[...])
    out_ref[...] = (jnp.dot(r.astype(bf), l3w_ref[...].astype(bf),
                            preferred_element_type=f32) + l3b_ref[...])


@functools.partial(jax.jit, static_argnames=())
def kernel(x, edge_index, W1, b1, W2, b2, gamma, beta, l1W, l1b, l2W, l2b,
           l3W, l3b):
    ei = edge_index.astype(jnp.int32)

    vmem = pl.BlockSpec(memory_space=pltpu.MemorySpace.VMEM)
    hbm = pl.BlockSpec(memory_space=pltpu.MemorySpace.HBM)
    out = pl.pallas_call(
        _gcn_kernel,
        out_shape=jax.ShapeDtypeStruct((1, 2), jnp.float32),
        in_specs=[vmem] * 8 + [hbm] + [vmem] * 5,
        out_specs=vmem,
        scratch_shapes=[
            pltpu.MemorySpace.VMEM((N_NODES, 64, 64), jnp.float32),
            pltpu.SemaphoreType.DMA,
        ],
    )(
        ei, x, W1, b1.reshape(1, -1), W2, b2.reshape(1, -1),
        gamma.reshape(1, -1), beta.reshape(1, -1),
        l1W.reshape(N_NODES, 64, 64), l1b.reshape(1, -1),
        l2W, l2b.reshape(1, -1), l3W, l3b.reshape(1, -1),
    )
    return out
